# baseline (speedup 1.0000x reference)
import functools

import jax
import jax.numpy as jnp
from jax import lax
from jax.experimental import pallas as pl
from jax.experimental.pallas import tpu as pltpu
from jax.experimental.pallas import tpu_sc as plsc

B = 1024
L = 200
D = 768
LANES = 16
DV = D // LANES
DW = D // 2
NG = D // 32
CHUNK = 40
CPR = L // CHUNK
NSLOTS = 2


def _tree_sum(vals):
    while len(vals) > 1:
        nxt = [vals[i] + vals[i + 1] for i in range(0, len(vals) - 1, 2)]
        if len(vals) % 2:
            nxt.append(vals[-1])
        vals = nxt
    return vals[0]


def _sc_body(nc, ns, wt_hbm, tok_hbm, b_hbm, out_hbm, sums_hbm,
             toks_v, buf0, buf1, acc_v, bias_v, sums_v, sg0, sg1):
    nw = nc * ns
    rows_w = B // nw
    tok_w = rows_w * L
    nchunk = tok_w // CHUNK

    cid = lax.axis_index("c")
    sid = lax.axis_index("s")
    wid = cid * ns + sid
    grow = wid * rows_w

    bufs = (buf0, buf1)
    sgs = (sg0, sg1)

    pltpu.sync_copy(tok_hbm.at[pl.ds(wid * tok_w, tok_w)], toks_v)
    pltpu.sync_copy(b_hbm, bias_v)

    def fill_row(r, carry):
        for c in range(DV):
            acc_v[r, pl.ds(c * LANES, LANES)] = bias_v[pl.ds(c * LANES, LANES)]
        return carry

    lax.fori_loop(0, rows_w, fill_row, 0)

    def start_gather(g, u):
        pltpu.async_copy(wt_hbm.at[toks_v.at[pl.ds(g * CHUNK, CHUNK)]],
                         bufs[u], sgs[u])

    for u in range(NSLOTS):
        start_gather(u, u)

    def go_body(go, carry):
        for u in range(NSLOTS):
            g = go * NSLOTS + u
            r = g // CPR
            pltpu.make_async_copy(
                wt_hbm.at[toks_v.at[pl.ds(g * CHUNK, CHUNK)]],
                bufs[u], sgs[u]).wait()

            def qbody(q, c2):
                qw = pl.multiple_of(q * LANES, LANES)
                qf = pl.multiple_of(q * 32, 32)
                words = [bufs[u][j, pl.ds(qw, LANES)] for j in range(CHUNK)]
                los = [lax.bitcast_convert_type(w << 16, jnp.float32) for w in words]
                his = [lax.bitcast_convert_type(w & jnp.int32(-65536),
                                                jnp.float32) for w in words]
                plsc.addupdate(acc_v.at[r, pl.ds(qf, LANES)], _tree_sum(los))
                plsc.addupdate(acc_v.at[r, pl.ds(qf + LANES, LANES)],
                               _tree_sum(his))
                return c2

            lax.fori_loop(0, NG, qbody, 0)

            @pl.when(g + NSLOTS < nchunk)
            def _():
                start_gather(g + NSLOTS, u)
        return carry

    lax.fori_loop(0, nchunk // NSLOTS, go_body, 0)

    def out_row(r, carry):
        s = acc_v[r, pl.ds(0, LANES)]
        for c in range(1, DV):
            s = s + acc_v[r, pl.ds(c * LANES, LANES)]
        sums_v[pl.ds(r * LANES, LANES)] = s
        return carry

    lax.fori_loop(0, rows_w, out_row, 0)

    pltpu.sync_copy(acc_v, out_hbm.at[pl.ds(grow, rows_w)])
    pltpu.sync_copy(sums_v, sums_hbm.at[pl.ds(wid * rows_w * LANES,
                                              rows_w * LANES)])


def kernel(token_ids, W, b):
    info = plsc.get_sparse_core_info()
    nc, ns = info.num_cores, info.num_subcores
    nw = nc * ns
    rows_w = B // nw
    assert (rows_w * L) % (CHUNK * NSLOTS) == 0

    vocab = W.shape[1]
    m = jnp.arange(32)
    fmap = jnp.where(m % 2 == 0, m // 2, LANES + (m - 1) // 2)
    p = jnp.arange(D)
    perm = (p // 32) * 32 + fmap[p % 32]
    wtb = W.T.astype(jnp.bfloat16)[:, perm]
    wt = lax.bitcast_convert_type(wtb.reshape(vocab, DW, 2), jnp.int32)
    toks = token_ids.reshape(-1).astype(jnp.int32)

    mesh = plsc.VectorSubcoreMesh(core_axis_name="c", subcore_axis_name="s")
    sc = pl.kernel(
        functools.partial(_sc_body, nc, ns),
        out_type=(
            jax.ShapeDtypeStruct((B, D), jnp.float32),
            jax.ShapeDtypeStruct((B * LANES,), jnp.float32),
        ),
        mesh=mesh,
        scratch_types=[
            pltpu.VMEM((rows_w * L,), jnp.int32),
            pltpu.VMEM((CHUNK, DW), jnp.int32),
            pltpu.VMEM((CHUNK, DW), jnp.int32),
            pltpu.VMEM((rows_w, D), jnp.float32),
            pltpu.VMEM((D,), jnp.float32),
            pltpu.VMEM((rows_w * LANES,), jnp.float32),
            pltpu.SemaphoreType.DMA,
            pltpu.SemaphoreType.DMA,
        ],
    )
    out2d, sums = sc(wt, toks, b)
    padding_mask = jnp.sum(sums.reshape(B, LANES), axis=1, keepdims=True) == 0.0
    return (out2d[:, None, :], padding_mask)


# speedup vs baseline: 1.1277x; 1.1277x over previous
import functools

import jax
import jax.numpy as jnp
from jax import lax
from jax.experimental import pallas as pl
from jax.experimental.pallas import tpu as pltpu
from jax.experimental.pallas import tpu_sc as plsc

B = 1024
L = 200
D = 768
LANES = 16
DV = D // LANES
DW = D // 2
NG = D // 32
CHUNK = 40
CPR = L // CHUNK
NSLOTS = 2


def _tree_sum(vals):
    while len(vals) > 1:
        nxt = [vals[i] + vals[i + 1] for i in range(0, len(vals) - 1, 2)]
        if len(vals) % 2:
            nxt.append(vals[-1])
        vals = nxt
    return vals[0]


def _sc_body(nc, ns, wt_hbm, tok_hbm, b_hbm, out_hbm, sums_hbm,
             toks_v, buf0, buf1, acc_v, bias_v, sums_v, sg0, sg1):
    nw = nc * ns
    rows_w = B // nw
    tok_w = rows_w * L
    nchunk = tok_w // CHUNK

    cid = lax.axis_index("c")
    sid = lax.axis_index("s")
    wid = cid * ns + sid
    grow = wid * rows_w

    bufs = (buf0, buf1)
    sgs = (sg0, sg1)

    pltpu.sync_copy(tok_hbm.at[pl.ds(wid * tok_w, tok_w)], toks_v)
    pltpu.sync_copy(b_hbm, bias_v)

    def fill_row(r, carry):
        for c in range(DV):
            acc_v[r, pl.ds(c * LANES, LANES)] = bias_v[pl.ds(c * LANES, LANES)]
        return carry

    lax.fori_loop(0, rows_w, fill_row, 0)

    def start_gather(g, u):
        pltpu.async_copy(wt_hbm.at[toks_v.at[pl.ds(g * CHUNK, CHUNK)]],
                         bufs[u], sgs[u])

    for u in range(NSLOTS):
        start_gather(u, u)

    def go_body(go, carry):
        for u in range(NSLOTS):
            g = go * NSLOTS + u
            r = g // CPR
            pltpu.make_async_copy(
                wt_hbm.at[toks_v.at[pl.ds(g * CHUNK, CHUNK)]],
                bufs[u], sgs[u]).wait()

            def qbody(q, c2):
                qw = pl.multiple_of(q * LANES, LANES)
                qf = pl.multiple_of(q * 32, 32)
                words = [bufs[u][j, pl.ds(qw, LANES)] for j in range(CHUNK)]
                los = [lax.bitcast_convert_type(w << 16, jnp.float32) for w in words]
                his = [lax.bitcast_convert_type(w & jnp.int32(-65536),
                                                jnp.float32) for w in words]
                plsc.addupdate(acc_v.at[r, pl.ds(qf, LANES)], _tree_sum(los))
                plsc.addupdate(acc_v.at[r, pl.ds(qf + LANES, LANES)],
                               _tree_sum(his))
                return c2

            lax.fori_loop(0, NG, qbody, 0)

            @pl.when(g + NSLOTS < nchunk)
            def _():
                start_gather(g + NSLOTS, u)
        return carry

    lax.fori_loop(0, nchunk // NSLOTS, go_body, 0)

    def out_row(r, carry):
        s = acc_v[r, pl.ds(0, LANES)]
        for c in range(1, DV):
            s = s + acc_v[r, pl.ds(c * LANES, LANES)]
        sums_v[pl.ds(r * LANES, LANES)] = s
        return carry

    lax.fori_loop(0, rows_w, out_row, 0)

    pltpu.sync_copy(acc_v, out_hbm.at[pl.ds(grow, rows_w)])
    pltpu.sync_copy(sums_v, sums_hbm.at[pl.ds(wid * rows_w * LANES,
                                              rows_w * LANES)])


def kernel(token_ids, W, b):
    info = plsc.get_sparse_core_info()
    nc, ns = info.num_cores, info.num_subcores
    nw = nc * ns
    rows_w = B // nw
    assert (rows_w * L) % (CHUNK * NSLOTS) == 0

    vocab = W.shape[1]
    wtb = W.astype(jnp.bfloat16).T
    wt = lax.bitcast_convert_type(wtb.reshape(vocab, DW, 2), jnp.int32)
    bias_g = b.reshape(D // 32, LANES, 2).transpose(0, 2, 1).reshape(D)
    toks = token_ids.reshape(-1).astype(jnp.int32)

    mesh = plsc.VectorSubcoreMesh(core_axis_name="c", subcore_axis_name="s")
    sc = pl.kernel(
        functools.partial(_sc_body, nc, ns),
        out_type=(
            jax.ShapeDtypeStruct((B, D), jnp.float32),
            jax.ShapeDtypeStruct((B * LANES,), jnp.float32),
        ),
        mesh=mesh,
        scratch_types=[
            pltpu.VMEM((rows_w * L,), jnp.int32),
            pltpu.VMEM((CHUNK, DW), jnp.int32),
            pltpu.VMEM((CHUNK, DW), jnp.int32),
            pltpu.VMEM((rows_w, D), jnp.float32),
            pltpu.VMEM((D,), jnp.float32),
            pltpu.VMEM((rows_w * LANES,), jnp.float32),
            pltpu.SemaphoreType.DMA,
            pltpu.SemaphoreType.DMA,
        ],
    )
    out_g, sums = sc(wt, toks, bias_g)
    out2d = out_g.reshape(B, D // 32, 2, LANES).transpose(0, 1, 3, 2)
    out2d = out2d.reshape(B, D)
    padding_mask = jnp.sum(sums.reshape(B, LANES), axis=1, keepdims=True) == 0.0
    return (out2d[:, None, :], padding_mask)


# speedup vs baseline: 6.8986x; 6.1173x over previous
import functools

import jax
import jax.numpy as jnp
from jax import lax
from jax.experimental import pallas as pl
from jax.experimental.pallas import tpu as pltpu
from jax.experimental.pallas import tpu_sc as plsc

B = 1024
L = 200
D = 768
LANES = 16
DV = D // LANES
CHUNK = 40
CPR = L // CHUNK
NSLOTS = 3


def _sc_body(nc, ns, wt_hbm, tok_hbm, b_hbm, out_hbm, sums_hbm,
             toks_v, buf0, buf1, buf2, acc_v, bias_v, sums_v, sg0, sg1, sg2):
    nw = nc * ns
    rows_w = B // nw
    tok_w = rows_w * L
    nchunk = tok_w // CHUNK

    cid = lax.axis_index("c")
    sid = lax.axis_index("s")
    wid = cid * ns + sid
    grow = wid * rows_w

    bufs = (buf0, buf1, buf2)
    sgs = (sg0, sg1, sg2)

    pltpu.sync_copy(tok_hbm.at[pl.ds(wid * tok_w, tok_w)], toks_v)
    pltpu.sync_copy(b_hbm, bias_v)

    def fill_row(r, carry):
        for c in range(DV):
            acc_v[r, pl.ds(c * LANES, LANES)] = bias_v[pl.ds(c * LANES, LANES)]
        return carry

    lax.fori_loop(0, rows_w, fill_row, 0)

    def start_gather(g, u):
        pltpu.async_copy(wt_hbm.at[toks_v.at[pl.ds(g * CHUNK, CHUNK)]],
                         bufs[u], sgs[u])

    for u in range(NSLOTS):
        start_gather(u, u)

    def go_body(go, carry):
        for u in range(NSLOTS):
            g = go * NSLOTS + u
            r = g // CPR
            pltpu.make_async_copy(
                wt_hbm.at[toks_v.at[pl.ds(g * CHUNK, CHUNK)]],
                bufs[u], sgs[u]).wait()

            def cbody(c, c2):
                co = pl.multiple_of(c * LANES, LANES)
                vals = [bufs[u][j, pl.ds(co, LANES)] for j in range(CHUNK)]
                while len(vals) > 1:
                    nxt = [vals[i] + vals[i + 1]
                           for i in range(0, len(vals) - 1, 2)]
                    if len(vals) % 2:
                        nxt.append(vals[-1])
                    vals = nxt
                plsc.addupdate(acc_v.at[r, pl.ds(co, LANES)], vals[0])
                return c2

            lax.fori_loop(0, DV, cbody, 0)

            @pl.when(g + NSLOTS < nchunk)
            def _():
                start_gather(g + NSLOTS, u)
        return carry

    lax.fori_loop(0, nchunk // NSLOTS, go_body, 0)

    for u in range(nchunk % NSLOTS):
        g = nchunk // NSLOTS * NSLOTS + u
        r = g // CPR
        pltpu.make_async_copy(
            wt_hbm.at[toks_v.at[pl.ds(g * CHUNK, CHUNK)]],
            bufs[u], sgs[u]).wait()

        def cbody_e(c, c2, _u=u, _r=r):
            co = pl.multiple_of(c * LANES, LANES)
            vals = [bufs[_u][j, pl.ds(co, LANES)] for j in range(CHUNK)]
            while len(vals) > 1:
                nxt = [vals[i] + vals[i + 1]
                       for i in range(0, len(vals) - 1, 2)]
                if len(vals) % 2:
                    nxt.append(vals[-1])
                vals = nxt
            plsc.addupdate(acc_v.at[_r, pl.ds(co, LANES)], vals[0])
            return c2

        lax.fori_loop(0, DV, cbody_e, 0)

    def out_row(r, carry):
        s = acc_v[r, pl.ds(0, LANES)]
        for c in range(1, DV):
            s = s + acc_v[r, pl.ds(c * LANES, LANES)]
        sums_v[pl.ds(r * LANES, LANES)] = s
        return carry

    lax.fori_loop(0, rows_w, out_row, 0)

    pltpu.sync_copy(acc_v, out_hbm.at[pl.ds(grow, rows_w)])
    pltpu.sync_copy(sums_v, sums_hbm.at[pl.ds(wid * rows_w * LANES,
                                              rows_w * LANES)])


def kernel(token_ids, W, b):
    info = plsc.get_sparse_core_info()
    nc, ns = info.num_cores, info.num_subcores
    nw = nc * ns
    rows_w = B // nw

    wt = W.T
    toks = token_ids.reshape(-1).astype(jnp.int32)

    mesh = plsc.VectorSubcoreMesh(core_axis_name="c", subcore_axis_name="s")
    sc = pl.kernel(
        functools.partial(_sc_body, nc, ns),
        out_type=(
            jax.ShapeDtypeStruct((B, D), jnp.float32),
            jax.ShapeDtypeStruct((B * LANES,), jnp.float32),
        ),
        mesh=mesh,
        scratch_types=[
            pltpu.VMEM((rows_w * L,), jnp.int32),
            pltpu.VMEM((CHUNK, D), jnp.float32),
            pltpu.VMEM((CHUNK, D), jnp.float32),
            pltpu.VMEM((CHUNK, D), jnp.float32),
            pltpu.VMEM((rows_w, D), jnp.float32),
            pltpu.VMEM((D,), jnp.float32),
            pltpu.VMEM((rows_w * LANES,), jnp.float32),
            pltpu.SemaphoreType.DMA,
            pltpu.SemaphoreType.DMA,
            pltpu.SemaphoreType.DMA,
        ],
    )
    out2d, sums = sc(wt, toks, b)
    padding_mask = jnp.sum(sums.reshape(B, LANES), axis=1, keepdims=True) == 0.0
    return (out2d[:, None, :], padding_mask)
